# baseline (speedup 1.0000x reference)
import jax
import jax.numpy as jnp
from jax import lax
from jax.experimental import pallas as pl
from jax.experimental.pallas import tpu as pltpu
from jax.experimental.pallas import tpu_sc as plsc

N = 10000
E = 160000
D = 256
H = 128
NC = 2
NS = 16
C = 80
K = (E // NS) // C
G = 25
R = 32
NCH = N // R
REM = N - NCH * R
NPASS = -(-NCH // NS)
HB = 80


def _finalize_chunk(c, nrows, cid, x_in, out, accv, xv, cntv, acc_sh,
                    cnt_sh):
    nb = c * R
    pltpu.sync_copy(acc_sh.at[pl.ds(nb, nrows)], accv.at[pl.ds(0, nrows)])
    pltpu.sync_copy(x_in.at[pl.ds(nb, nrows), pl.ds(cid * H, H)],
                    xv.at[pl.ds(0, nrows)])
    pltpu.sync_copy(cnt_sh.at[lax.shift_right_logical(nb, 7), 0], cntv)

    lanes0 = jnp.zeros((16,), jnp.int32)
    coff = nb & (H - 1)

    def fin(i, _):
        cnt = plsc.load_gather(cntv, [lanes0 + (coff + i)])
        rden = 1.0 / jnp.maximum(cnt, 1.0)
        mask = cnt > 0.5
        for j in range(H // 16):
            accv[i, pl.ds(j * 16, 16)] = accv[i, pl.ds(j * 16, 16)] * rden
            xx = xv[i, pl.ds(j * 16, 16)]
            xv[i, pl.ds(j * 16, 16)] = jnp.where(mask, xx, 0.0)
        return _
    lax.fori_loop(0, nrows, fin, 0)

    pltpu.sync_copy(accv.at[pl.ds(0, nrows)],
                    out.at[pl.ds(nb, nrows), pl.ds(cid * H, H)])
    pltpu.sync_copy(xv.at[pl.ds(0, nrows)],
                    out.at[pl.ds(nb, nrows), pl.ds(D + cid * H, H)])


def _body(xt, x_in, rows2, col2, out, ridx, cidx, rowsb, accv, xv,
          cntb, cntv, idv, acc_sh, cnt_sh, sems, ssems):
    cid = lax.axis_index("c")
    sid = lax.axis_index("s")

    def zacc(i, _):
        for j in range(H // 16):
            accv[i, pl.ds(j * 16, 16)] = jnp.zeros((16,), jnp.float32)
        return _
    lax.fori_loop(0, R, zacc, 0)

    def zcnt(i, _):
        for j in range(H // 16):
            cntb[i, 0, pl.ds(j * 16, 16)] = jnp.zeros((16,), jnp.float32)
        return _
    lax.fori_loop(0, HB, zcnt, 0)

    for j in range(HB // 16):
        idv[pl.ds(j * 16, 16)] = lax.iota(jnp.int32, 16) + j * 16

    for q in range(NPASS):
        c = q * NS + sid

        @pl.when(c < NCH)
        def _():
            pltpu.sync_copy(accv, acc_sh.at[pl.ds(c * R, R)])

    @pl.when(sid == NS - 1)
    def _():
        pltpu.sync_copy(accv.at[pl.ds(0, REM)],
                        acc_sh.at[pl.ds(NCH * R, REM)])

    @pl.when(sid == 0)
    def _():
        pltpu.sync_copy(cntb, cnt_sh)

    pcnt, _pm = plsc.scan_count(idv[pl.ds(0, 16)])
    bias = 1 - jnp.max(pcnt)

    plsc.subcore_barrier()

    for g in range(K // G):
        pltpu.sync_copy(rows2.at[cid, sid, pl.ds(g * G, G)], ridx)
        pltpu.sync_copy(col2.at[sid, pl.ds(g * G, G)], cidx)
        pltpu.async_copy(xt.at[ridx.at[0, 0]], rowsb.at[0], sems.at[0])

        def step(k, carry):
            b1 = (k + 1) & 1

            @pl.when(k >= 1)
            def _():
                pltpu.make_async_copy(
                    rowsb.at[b1], acc_sh.at[cidx.at[k - 1, 0]],
                    ssems.at[b1]).wait()

            @pl.when(k < G - 1)
            def _():
                pltpu.async_copy(xt.at[ridx.at[k + 1, 0]], rowsb.at[b1],
                                 sems.at[b1])

            for j in range(C // 16):
                cv = cidx[k, 0, pl.ds(j * 16, 16)]
                cnt, last = plsc.scan_count(cv)
                val = (cnt + bias).astype(jnp.float32)
                plsc.addupdate_scatter(
                    cntb, [lax.shift_right_logical(cv, 7),
                           jnp.zeros((16,), jnp.int32),
                           cv & (H - 1)], val, mask=last)

            b = k & 1
            pltpu.make_async_copy(xt.at[ridx.at[k, 0]], rowsb.at[b],
                                  sems.at[b]).wait()
            pltpu.async_copy(rowsb.at[b], acc_sh.at[cidx.at[k, 0]],
                             ssems.at[b], add=True)
            return carry
        lax.fori_loop(0, G, step, 0)

        pltpu.make_async_copy(rowsb.at[(G - 1) & 1],
                              acc_sh.at[cidx.at[G - 1, 0]],
                              ssems.at[(G - 1) & 1]).wait()

    pltpu.sync_copy(cntb, cnt_sh.at[idv], add=True)

    plsc.subcore_barrier()

    for q in range(NPASS):
        c = q * NS + sid

        @pl.when(c < NCH)
        def _():
            _finalize_chunk(c, R, cid, x_in, out, accv, xv, cntv,
                            acc_sh, cnt_sh)

    @pl.when(sid == NS - 1)
    def _():
        _finalize_chunk(NCH, REM, cid, x_in, out, accv, xv, cntv,
                        acc_sh, cnt_sh)


@jax.jit
def kernel(x, es):
    es = es.astype(jnp.int32)
    col = es[0]
    row = es[1]
    xt = x.reshape(2 * N, H)
    r2 = 2 * row
    rows2 = jnp.stack([r2, r2 + 1]).reshape(NC, NS, K, 1, C)
    col2 = col.reshape(NS, K, 1, C)

    mesh = plsc.VectorSubcoreMesh(core_axis_name="c", subcore_axis_name="s")
    f = pl.kernel(
        _body,
        out_type=jax.ShapeDtypeStruct((N, 2 * D), jnp.float32),
        mesh=mesh,
        compiler_params=pltpu.CompilerParams(needs_layout_passes=False),
        scratch_types=[
            pltpu.VMEM((G, 1, C), jnp.int32),
            pltpu.VMEM((G, 1, C), jnp.int32),
            pltpu.VMEM((2, C, H), jnp.float32),
            pltpu.VMEM((R, H), jnp.float32),
            pltpu.VMEM((R, H), jnp.float32),
            pltpu.VMEM((HB, 1, H), jnp.float32),
            pltpu.VMEM((H,), jnp.float32),
            pltpu.VMEM((HB,), jnp.int32),
            pltpu.VMEM_SHARED((N, H), jnp.float32),
            pltpu.VMEM_SHARED((HB, 1, H), jnp.float32),
            pltpu.SemaphoreType.DMA((2,)),
            pltpu.SemaphoreType.DMA((2,)),
        ],
    )
    return f(xt, x, rows2, col2)


# speedup vs baseline: 1.1047x; 1.1047x over previous
import jax
import jax.numpy as jnp
from jax import lax
from jax.experimental import pallas as pl
from jax.experimental.pallas import tpu as pltpu
from jax.experimental.pallas import tpu_sc as plsc

N = 10000
E = 160000
D = 256
H = 128
NC = 2
NS = 16
C = 80
K = (E // NS) // C
G = 25
R = 64
NCH = N // R
REM = N - NCH * R
NPASS = -(-NCH // NS)
HB = 80


def _finalize_chunk(c, nrows, cid, x_in, out, accv, xv, cntv, acc_sh,
                    cnt_sh, fsems):
    nb = c * R
    ra = pltpu.async_copy(acc_sh.at[pl.ds(nb, nrows)],
                          accv.at[pl.ds(0, nrows)], fsems.at[0])
    rx = pltpu.async_copy(x_in.at[pl.ds(nb, nrows), pl.ds(cid * H, H)],
                          xv.at[pl.ds(0, nrows)], fsems.at[1])
    rc = pltpu.async_copy(cnt_sh.at[lax.shift_right_logical(nb, 7), 0],
                          cntv, fsems.at[2])
    ra.wait()
    rx.wait()
    rc.wait()

    lanes0 = jnp.zeros((16,), jnp.int32)
    coff = nb & (H - 1)

    def fin16(i16, _):
        cnt16 = cntv[pl.ds(coff + i16 * 16, 16)]
        rden16 = 1.0 / jnp.maximum(cnt16, 1.0)
        mke16 = jnp.minimum(cnt16, 1.0)

        def fin(i, __):
            bi = lanes0 + i
            rden = rden16[bi]
            mke = mke16[bi]
            row = i16 * 16 + i
            for j in range(H // 16):
                accv[row, pl.ds(j * 16, 16)] = (
                    accv[row, pl.ds(j * 16, 16)] * rden)
                xv[row, pl.ds(j * 16, 16)] = (
                    xv[row, pl.ds(j * 16, 16)] * mke)
            return __
        lax.fori_loop(0, 16, fin, 0)
        return _
    lax.fori_loop(0, nrows // 16, fin16, 0)

    wa = pltpu.async_copy(accv.at[pl.ds(0, nrows)],
                          out.at[pl.ds(nb, nrows), pl.ds(cid * H, H)],
                          fsems.at[0])
    wx = pltpu.async_copy(xv.at[pl.ds(0, nrows)],
                          out.at[pl.ds(nb, nrows), pl.ds(D + cid * H, H)],
                          fsems.at[1])
    wa.wait()
    wx.wait()


def _body(xt, x_in, rows2, col2, out, ridx, cidx, rowsb,
          cntb, cntv, idv, acc_sh, cnt_sh, sems, ssems, fsems):
    cid = lax.axis_index("c")
    sid = lax.axis_index("s")
    accv = rowsb.at[1]
    xv = rowsb.at[0]

    def zacc(i, _):
        for j in range(H // 16):
            accv[i, pl.ds(j * 16, 16)] = jnp.zeros((16,), jnp.float32)
        return _
    lax.fori_loop(0, R, zacc, 0)

    def zcnt(i, _):
        for j in range(H // 16):
            cntb[i, 0, pl.ds(j * 16, 16)] = jnp.zeros((16,), jnp.float32)
        return _
    lax.fori_loop(0, HB, zcnt, 0)

    for j in range(HB // 16):
        idv[pl.ds(j * 16, 16)] = lax.iota(jnp.int32, 16) + j * 16

    for q in range(NPASS):
        c = q * NS + sid

        @pl.when(c < NCH)
        def _():
            pltpu.async_copy(accv.at[pl.ds(0, R)],
                             acc_sh.at[pl.ds(c * R, R)], fsems.at[0])

    @pl.when(sid == NS - 1)
    def _():
        pltpu.async_copy(accv.at[pl.ds(0, REM)],
                         acc_sh.at[pl.ds(NCH * R, REM)], fsems.at[1])

    @pl.when(sid == 0)
    def _():
        pltpu.async_copy(cntb, cnt_sh, fsems.at[2])

    for q in range(NPASS):
        c = q * NS + sid

        @pl.when(c < NCH)
        def _():
            pltpu.make_async_copy(accv.at[pl.ds(0, R)],
                                  acc_sh.at[pl.ds(c * R, R)],
                                  fsems.at[0]).wait()

    @pl.when(sid == NS - 1)
    def _():
        pltpu.make_async_copy(accv.at[pl.ds(0, REM)],
                              acc_sh.at[pl.ds(NCH * R, REM)],
                              fsems.at[1]).wait()

    @pl.when(sid == 0)
    def _():
        pltpu.make_async_copy(cntb, cnt_sh, fsems.at[2]).wait()

    pcnt, _pm = plsc.scan_count(idv[pl.ds(0, 16)])
    bias = 1 - jnp.max(pcnt)

    plsc.subcore_barrier()

    for g in range(K // G):
        pltpu.sync_copy(rows2.at[cid, sid, pl.ds(g * G, G)], ridx)
        pltpu.sync_copy(col2.at[sid, pl.ds(g * G, G)], cidx)
        pltpu.async_copy(xt.at[ridx.at[0, 0]], rowsb.at[0], sems.at[0])

        def step(k, carry):
            b1 = (k + 1) & 1

            @pl.when(k >= 1)
            def _():
                pltpu.make_async_copy(
                    rowsb.at[b1], acc_sh.at[cidx.at[k - 1, 0]],
                    ssems.at[b1]).wait()

            @pl.when(k < G - 1)
            def _():
                pltpu.async_copy(xt.at[ridx.at[k + 1, 0]], rowsb.at[b1],
                                 sems.at[b1])

            for j in range(C // 16):
                cv = cidx[k, 0, pl.ds(j * 16, 16)]
                cnt, last = plsc.scan_count(cv)
                val = (cnt + bias).astype(jnp.float32)
                plsc.addupdate_scatter(
                    cntb, [lax.shift_right_logical(cv, 7),
                           jnp.zeros((16,), jnp.int32),
                           cv & (H - 1)], val, mask=last)

            b = k & 1
            pltpu.make_async_copy(xt.at[ridx.at[k, 0]], rowsb.at[b],
                                  sems.at[b]).wait()
            pltpu.async_copy(rowsb.at[b], acc_sh.at[cidx.at[k, 0]],
                             ssems.at[b], add=True)
            return carry
        lax.fori_loop(0, G, step, 0)

        pltpu.make_async_copy(rowsb.at[(G - 1) & 1],
                              acc_sh.at[cidx.at[G - 1, 0]],
                              ssems.at[(G - 1) & 1]).wait()

    pltpu.sync_copy(cntb, cnt_sh.at[idv], add=True)

    plsc.subcore_barrier()

    for q in range(NPASS):
        c = q * NS + sid

        @pl.when(c < NCH)
        def _():
            _finalize_chunk(c, R, cid, x_in, out, accv, xv, cntv,
                            acc_sh, cnt_sh, fsems)

    @pl.when(sid == NS - 1)
    def _():
        _finalize_chunk(NCH, REM, cid, x_in, out, accv, xv, cntv,
                        acc_sh, cnt_sh, fsems)


@jax.jit
def kernel(x, es):
    es = es.astype(jnp.int32)
    col = es[0]
    row = es[1]
    xt = x.reshape(2 * N, H)
    r2 = 2 * row
    rows2 = jnp.stack([r2, r2 + 1]).reshape(NC, NS, K, 1, C)
    col2 = col.reshape(NS, K, 1, C)

    mesh = plsc.VectorSubcoreMesh(core_axis_name="c", subcore_axis_name="s")
    f = pl.kernel(
        _body,
        out_type=jax.ShapeDtypeStruct((N, 2 * D), jnp.float32),
        mesh=mesh,
        compiler_params=pltpu.CompilerParams(needs_layout_passes=False),
        scratch_types=[
            pltpu.VMEM((G, 1, C), jnp.int32),
            pltpu.VMEM((G, 1, C), jnp.int32),
            pltpu.VMEM((2, C, H), jnp.float32),
            pltpu.VMEM((HB, 1, H), jnp.float32),
            pltpu.VMEM((H,), jnp.float32),
            pltpu.VMEM((HB,), jnp.int32),
            pltpu.VMEM_SHARED((N, H), jnp.float32),
            pltpu.VMEM_SHARED((HB, 1, H), jnp.float32),
            pltpu.SemaphoreType.DMA((2,)),
            pltpu.SemaphoreType.DMA((2,)),
            pltpu.SemaphoreType.DMA((3,)),
        ],
    )
    return f(xt, x, rows2, col2)


# speedup vs baseline: 1.1429x; 1.0346x over previous
import jax
import jax.numpy as jnp
from jax import lax
from jax.experimental import pallas as pl
from jax.experimental.pallas import tpu as pltpu
from jax.experimental.pallas import tpu_sc as plsc

N = 10000
E = 160000
D = 256
H = 128
NC = 2
NS = 16
C = 80
K = (E // NS) // C
G = 25
R = 64
NCH = N // R
REM = N - NCH * R
NPASS = -(-NCH // NS)
HB = 80


def _finalize_chunk(c, nrows, cid, x_in, out, accv, xv, cntv, acc_sh,
                    cnt_sh, fsems):
    nb = c * R
    ra = pltpu.async_copy(acc_sh.at[pl.ds(nb, nrows)],
                          accv.at[pl.ds(0, nrows)], fsems.at[0])
    rx = pltpu.async_copy(x_in.at[pl.ds(nb, nrows), pl.ds(cid * H, H)],
                          xv.at[pl.ds(0, nrows)], fsems.at[1])
    rc = pltpu.async_copy(cnt_sh.at[lax.shift_right_logical(nb, 7), 0],
                          cntv, fsems.at[2])
    ra.wait()
    rx.wait()
    rc.wait()

    lanes0 = jnp.zeros((16,), jnp.int32)
    coff = nb & (H - 1)

    def fin16(i16, _):
        cnt16 = cntv[pl.ds(coff + i16 * 16, 16)]
        rden16 = 1.0 / jnp.maximum(cnt16, 1.0)
        mke16 = jnp.minimum(cnt16, 1.0)

        def fin(i, __):
            bi = lanes0 + i
            rden = rden16[bi]
            mke = mke16[bi]
            row = i16 * 16 + i
            for j in range(H // 16):
                accv[row, pl.ds(j * 16, 16)] = (
                    accv[row, pl.ds(j * 16, 16)] * rden)
                xv[row, pl.ds(j * 16, 16)] = (
                    xv[row, pl.ds(j * 16, 16)] * mke)
            return __
        lax.fori_loop(0, 16, fin, 0)
        return _
    lax.fori_loop(0, nrows // 16, fin16, 0)

    wa = pltpu.async_copy(accv.at[pl.ds(0, nrows)],
                          out.at[pl.ds(nb, nrows), pl.ds(cid * H, H)],
                          fsems.at[0])
    wx = pltpu.async_copy(xv.at[pl.ds(0, nrows)],
                          out.at[pl.ds(nb, nrows), pl.ds(D + cid * H, H)],
                          fsems.at[1])
    wa.wait()
    wx.wait()


def _body(xt, x_in, rows2, col2, out, ridx, cidx, rowsb,
          cntb, cntv, idv, acc_sh, cnt_sh, sems, ssems, fsems, isems):
    cid = lax.axis_index("c")
    sid = lax.axis_index("s")
    accv = rowsb.at[1]
    xv = rowsb.at[0]

    def zacc(i, _):
        for j in range(H // 16):
            accv[i, pl.ds(j * 16, 16)] = jnp.zeros((16,), jnp.float32)
        return _
    lax.fori_loop(0, R, zacc, 0)

    def zcnt(i, _):
        for j in range(H // 16):
            cntb[i, 0, pl.ds(j * 16, 16)] = jnp.zeros((16,), jnp.float32)
        return _
    lax.fori_loop(0, HB, zcnt, 0)

    for j in range(HB // 16):
        idv[pl.ds(j * 16, 16)] = lax.iota(jnp.int32, 16) + j * 16

    for q in range(NPASS):
        c = q * NS + sid

        @pl.when(c < NCH)
        def _():
            pltpu.async_copy(accv.at[pl.ds(0, R)],
                             acc_sh.at[pl.ds(c * R, R)], fsems.at[0])

    @pl.when(sid == NS - 1)
    def _():
        pltpu.async_copy(accv.at[pl.ds(0, REM)],
                         acc_sh.at[pl.ds(NCH * R, REM)], fsems.at[1])

    @pl.when(sid == 0)
    def _():
        pltpu.async_copy(cntb, cnt_sh, fsems.at[2])

    pltpu.async_copy(rows2.at[cid, sid, pl.ds(0, G)], ridx.at[0],
                     isems.at[0])
    pltpu.async_copy(col2.at[sid, pl.ds(0, G)], cidx.at[0], isems.at[0])

    for q in range(NPASS):
        c = q * NS + sid

        @pl.when(c < NCH)
        def _():
            pltpu.make_async_copy(accv.at[pl.ds(0, R)],
                                  acc_sh.at[pl.ds(c * R, R)],
                                  fsems.at[0]).wait()

    @pl.when(sid == NS - 1)
    def _():
        pltpu.make_async_copy(accv.at[pl.ds(0, REM)],
                              acc_sh.at[pl.ds(NCH * R, REM)],
                              fsems.at[1]).wait()

    @pl.when(sid == 0)
    def _():
        pltpu.make_async_copy(cntb, cnt_sh, fsems.at[2]).wait()

    pcnt, _pm = plsc.scan_count(idv[pl.ds(0, 16)])
    bias = 1 - jnp.max(pcnt)

    plsc.subcore_barrier()

    for g in range(K // G):
        gb = g & 1
        pltpu.make_async_copy(rows2.at[cid, sid, pl.ds(g * G, G)],
                              ridx.at[gb], isems.at[gb]).wait()
        pltpu.make_async_copy(col2.at[sid, pl.ds(g * G, G)],
                              cidx.at[gb], isems.at[gb]).wait()
        if g + 1 < K // G:
            gb1 = (g + 1) & 1
            pltpu.async_copy(rows2.at[cid, sid, pl.ds((g + 1) * G, G)],
                             ridx.at[gb1], isems.at[gb1])
            pltpu.async_copy(col2.at[sid, pl.ds((g + 1) * G, G)],
                             cidx.at[gb1], isems.at[gb1])
        pltpu.async_copy(xt.at[ridx.at[gb, 0, 0]], rowsb.at[0], sems.at[0])

        def step(k, carry):
            b1 = (k + 1) & 1

            @pl.when(k >= 1)
            def _():
                pltpu.make_async_copy(
                    rowsb.at[b1], acc_sh.at[cidx.at[gb, k - 1, 0]],
                    ssems.at[b1]).wait()

            @pl.when(k < G - 1)
            def _():
                pltpu.async_copy(xt.at[ridx.at[gb, k + 1, 0]],
                                 rowsb.at[b1], sems.at[b1])

            for j in range(C // 16):
                cv = cidx[gb, k, 0, pl.ds(j * 16, 16)]
                cnt, last = plsc.scan_count(cv)
                val = (cnt + bias).astype(jnp.float32)
                plsc.addupdate_scatter(
                    cntb, [lax.shift_right_logical(cv, 7),
                           jnp.zeros((16,), jnp.int32),
                           cv & (H - 1)], val, mask=last)

            b = k & 1
            pltpu.make_async_copy(xt.at[ridx.at[gb, k, 0]], rowsb.at[b],
                                  sems.at[b]).wait()
            pltpu.async_copy(rowsb.at[b], acc_sh.at[cidx.at[gb, k, 0]],
                             ssems.at[b], add=True)
            return carry
        lax.fori_loop(0, G, step, 0)

        pltpu.make_async_copy(rowsb.at[(G - 1) & 1],
                              acc_sh.at[cidx.at[gb, G - 1, 0]],
                              ssems.at[(G - 1) & 1]).wait()

    pltpu.sync_copy(cntb, cnt_sh.at[idv], add=True)

    plsc.subcore_barrier()

    for q in range(NPASS):
        c = q * NS + sid

        @pl.when(c < NCH)
        def _():
            _finalize_chunk(c, R, cid, x_in, out, accv, xv, cntv,
                            acc_sh, cnt_sh, fsems)

    @pl.when(sid == NS - 1)
    def _():
        _finalize_chunk(NCH, REM, cid, x_in, out, accv, xv, cntv,
                        acc_sh, cnt_sh, fsems)


@jax.jit
def kernel(x, es):
    es = es.astype(jnp.int32)
    col = es[0]
    row = es[1]
    xt = x.reshape(2 * N, H)
    r2 = 2 * row
    rows2 = jnp.stack([r2, r2 + 1]).reshape(NC, NS, K, 1, C)
    col2 = col.reshape(NS, K, 1, C)

    mesh = plsc.VectorSubcoreMesh(core_axis_name="c", subcore_axis_name="s")
    f = pl.kernel(
        _body,
        out_type=jax.ShapeDtypeStruct((N, 2 * D), jnp.float32),
        mesh=mesh,
        compiler_params=pltpu.CompilerParams(needs_layout_passes=False),
        scratch_types=[
            pltpu.VMEM((2, G, 1, C), jnp.int32),
            pltpu.VMEM((2, G, 1, C), jnp.int32),
            pltpu.VMEM((2, C, H), jnp.float32),
            pltpu.VMEM((HB, 1, H), jnp.float32),
            pltpu.VMEM((H,), jnp.float32),
            pltpu.VMEM((HB,), jnp.int32),
            pltpu.VMEM_SHARED((N, H), jnp.float32),
            pltpu.VMEM_SHARED((HB, 1, H), jnp.float32),
            pltpu.SemaphoreType.DMA((2,)),
            pltpu.SemaphoreType.DMA((2,)),
            pltpu.SemaphoreType.DMA((3,)),
            pltpu.SemaphoreType.DMA((2,)),
        ],
    )
    return f(xt, x, rows2, col2)


# speedup vs baseline: 1.1808x; 1.0332x over previous
import jax
import jax.numpy as jnp
from jax import lax
from jax.experimental import pallas as pl
from jax.experimental.pallas import tpu as pltpu
from jax.experimental.pallas import tpu_sc as plsc

N = 10000
E = 160000
D = 256
H = 128
NC = 2
NS = 16
C = 80
K = (E // NS) // C
G = 25
R = 32
NCH = N // R
REM = N - NCH * R
NPASS = -(-NCH // NS)
RZ = 64
NCHZ = N // RZ
REMZ = N - NCHZ * RZ
NPASSZ = -(-NCHZ // NS)
HB = 80


def _fz_reads(c, nrows, p, cid, x_in, accv, xv, cntv, acc_sh, cnt_sh,
              rsems, make):
    nb = c * R
    o = p * R
    f = pltpu.make_async_copy if make else (
        lambda s, d, m: pltpu.async_copy(s, d, m))
    da = f(acc_sh.at[pl.ds(nb, nrows)], accv.at[pl.ds(o, nrows)],
           rsems.at[p, 0])
    dx = f(x_in.at[pl.ds(nb, nrows), pl.ds(cid * H, H)],
           xv.at[pl.ds(o, nrows)], rsems.at[p, 1])
    dc = f(cnt_sh.at[lax.shift_right_logical(nb, 7), 0], cntv.at[p, 0],
           rsems.at[p, 2])
    return da, dx, dc


def _fz_writes(c, nrows, p, cid, out, accv, xv, wsems, make):
    nb = c * R
    o = p * R
    f = pltpu.make_async_copy if make else (
        lambda s, d, m: pltpu.async_copy(s, d, m))
    wa = f(accv.at[pl.ds(o, nrows)],
           out.at[pl.ds(nb, nrows), pl.ds(cid * H, H)], wsems.at[p, 0])
    wx = f(xv.at[pl.ds(o, nrows)],
           out.at[pl.ds(nb, nrows), pl.ds(D + cid * H, H)],
           wsems.at[p, 1])
    return wa, wx


def _fz_compute(c, nrows, p, accv, xv, cntv):
    nb = c * R
    lanes0 = jnp.zeros((16,), jnp.int32)
    coff = nb & (H - 1)
    o = p * R

    def fin16(i16, _):
        cnt16 = cntv[p, 0, pl.ds(coff + i16 * 16, 16)]
        rden16 = 1.0 / jnp.maximum(cnt16, 1.0)
        mke16 = jnp.minimum(cnt16, 1.0)

        def fin(i, __):
            bi = lanes0 + i
            rden = rden16[bi]
            mke = mke16[bi]
            row = o + i16 * 16 + i
            for j in range(H // 16):
                accv[row, pl.ds(j * 16, 16)] = (
                    accv[row, pl.ds(j * 16, 16)] * rden)
                xv[row, pl.ds(j * 16, 16)] = (
                    xv[row, pl.ds(j * 16, 16)] * mke)
            return __
        lax.fori_loop(0, 16, fin, 0)
        return _
    lax.fori_loop(0, nrows // 16, fin16, 0)


def _body(xt, x_in, rows2, col2, out, ridx, cidx, rowsb,
          cntb, cntv, idv, acc_sh, cnt_sh, sems, ssems, fsems, isems,
          rsems, wsems):
    cid = lax.axis_index("c")
    sid = lax.axis_index("s")
    accv = rowsb.at[1]
    xv = rowsb.at[0]

    def zacc(i, _):
        for j in range(H // 16):
            accv[i, pl.ds(j * 16, 16)] = jnp.zeros((16,), jnp.float32)
        return _
    lax.fori_loop(0, RZ, zacc, 0)

    def zcnt(i, _):
        for j in range(H // 16):
            cntb[i, 0, pl.ds(j * 16, 16)] = jnp.zeros((16,), jnp.float32)
        return _
    lax.fori_loop(0, HB, zcnt, 0)

    for j in range(HB // 16):
        idv[pl.ds(j * 16, 16)] = lax.iota(jnp.int32, 16) + j * 16

    for q in range(NPASSZ):
        c = q * NS + sid

        @pl.when(c < NCHZ)
        def _():
            pltpu.async_copy(accv.at[pl.ds(0, RZ)],
                             acc_sh.at[pl.ds(c * RZ, RZ)], fsems.at[0])

    @pl.when(sid == NS - 1)
    def _():
        pltpu.async_copy(accv.at[pl.ds(0, REMZ)],
                         acc_sh.at[pl.ds(NCHZ * RZ, REMZ)], fsems.at[1])

    @pl.when(sid == 0)
    def _():
        pltpu.async_copy(cntb, cnt_sh, fsems.at[2])

    pltpu.async_copy(rows2.at[cid, sid, pl.ds(0, G)], ridx.at[0],
                     isems.at[0])
    pltpu.async_copy(col2.at[sid, pl.ds(0, G)], cidx.at[0], isems.at[0])

    for q in range(NPASSZ):
        c = q * NS + sid

        @pl.when(c < NCHZ)
        def _():
            pltpu.make_async_copy(accv.at[pl.ds(0, RZ)],
                                  acc_sh.at[pl.ds(c * RZ, RZ)],
                                  fsems.at[0]).wait()

    @pl.when(sid == NS - 1)
    def _():
        pltpu.make_async_copy(accv.at[pl.ds(0, REMZ)],
                              acc_sh.at[pl.ds(NCHZ * RZ, REMZ)],
                              fsems.at[1]).wait()

    @pl.when(sid == 0)
    def _():
        pltpu.make_async_copy(cntb, cnt_sh, fsems.at[2]).wait()

    pcnt, _pm = plsc.scan_count(idv[pl.ds(0, 16)])
    bias = 1 - jnp.max(pcnt)

    plsc.subcore_barrier()

    for g in range(K // G):
        gb = g & 1
        pltpu.make_async_copy(rows2.at[cid, sid, pl.ds(g * G, G)],
                              ridx.at[gb], isems.at[gb]).wait()
        pltpu.make_async_copy(col2.at[sid, pl.ds(g * G, G)],
                              cidx.at[gb], isems.at[gb]).wait()
        if g + 1 < K // G:
            gb1 = (g + 1) & 1
            pltpu.async_copy(rows2.at[cid, sid, pl.ds((g + 1) * G, G)],
                             ridx.at[gb1], isems.at[gb1])
            pltpu.async_copy(col2.at[sid, pl.ds((g + 1) * G, G)],
                             cidx.at[gb1], isems.at[gb1])
        pltpu.async_copy(xt.at[ridx.at[gb, 0, 0]], rowsb.at[0], sems.at[0])

        def step(k, carry):
            b1 = (k + 1) & 1

            @pl.when(k >= 1)
            def _():
                pltpu.make_async_copy(
                    rowsb.at[b1], acc_sh.at[cidx.at[gb, k - 1, 0]],
                    ssems.at[b1]).wait()

            @pl.when(k < G - 1)
            def _():
                pltpu.async_copy(xt.at[ridx.at[gb, k + 1, 0]],
                                 rowsb.at[b1], sems.at[b1])

            for j in range(C // 16):
                cv = cidx[gb, k, 0, pl.ds(j * 16, 16)]
                cnt, last = plsc.scan_count(cv)
                val = (cnt + bias).astype(jnp.float32)
                plsc.addupdate_scatter(
                    cntb, [lax.shift_right_logical(cv, 7),
                           jnp.zeros((16,), jnp.int32),
                           cv & (H - 1)], val, mask=last)

            b = k & 1
            pltpu.make_async_copy(xt.at[ridx.at[gb, k, 0]], rowsb.at[b],
                                  sems.at[b]).wait()
            pltpu.async_copy(rowsb.at[b], acc_sh.at[cidx.at[gb, k, 0]],
                             ssems.at[b], add=True)
            return carry
        lax.fori_loop(0, G, step, 0)

        pltpu.make_async_copy(rowsb.at[(G - 1) & 1],
                              acc_sh.at[cidx.at[gb, G - 1, 0]],
                              ssems.at[(G - 1) & 1]).wait()

    pltpu.sync_copy(cntb, cnt_sh.at[idv], add=True)

    plsc.subcore_barrier()

    _fz_reads(sid, R, 0, cid, x_in, accv, xv, cntv, acc_sh, cnt_sh,
              rsems, False)

    def fzstep(q, carry):
        c = q * NS + sid
        p = q & 1
        p1 = 1 - p

        @pl.when(jnp.logical_and(q >= 1, c - NS < NCH))
        def _():
            for w in _fz_writes(c - NS, R, p1, cid, out, accv, xv,
                                wsems, True):
                w.wait()

        @pl.when(jnp.logical_and(q + 1 < NPASS, c + NS < NCH))
        def _():
            _fz_reads(c + NS, R, p1, cid, x_in, accv, xv, cntv, acc_sh,
                      cnt_sh, rsems, False)

        @pl.when(c < NCH)
        def _():
            for d in _fz_reads(c, R, p, cid, x_in, accv, xv, cntv,
                               acc_sh, cnt_sh, rsems, True):
                d.wait()
            _fz_compute(c, R, p, accv, xv, cntv)
            _fz_writes(c, R, p, cid, out, accv, xv, wsems, False)
        return carry
    lax.fori_loop(0, NPASS, fzstep, 0)

    qL = NPASS - 1
    cL = qL * NS + sid

    @pl.when(cL < NCH)
    def _():
        for w in _fz_writes(cL, R, qL & 1, cid, out, accv, xv, wsems,
                            True):
            w.wait()

    @pl.when(sid == NS - 1)
    def _():
        _fz_reads(NCH, REM, 0, cid, x_in, accv, xv, cntv, acc_sh,
                  cnt_sh, rsems, False)
        for d in _fz_reads(NCH, REM, 0, cid, x_in, accv, xv, cntv,
                           acc_sh, cnt_sh, rsems, True):
            d.wait()
        _fz_compute(NCH, REM, 0, accv, xv, cntv)
        _fz_writes(NCH, REM, 0, cid, out, accv, xv, wsems, False)
        for w in _fz_writes(NCH, REM, 0, cid, out, accv, xv, wsems,
                            True):
            w.wait()


@jax.jit
def kernel(x, es):
    es = es.astype(jnp.int32)
    col = es[0]
    row = es[1]
    xt = x.reshape(2 * N, H)
    r2 = 2 * row
    rows2 = jnp.stack([r2, r2 + 1]).reshape(NC, NS, K, 1, C)
    col2 = col.reshape(NS, K, 1, C)

    mesh = plsc.VectorSubcoreMesh(core_axis_name="c", subcore_axis_name="s")
    f = pl.kernel(
        _body,
        out_type=jax.ShapeDtypeStruct((N, 2 * D), jnp.float32),
        mesh=mesh,
        compiler_params=pltpu.CompilerParams(needs_layout_passes=False),
        scratch_types=[
            pltpu.VMEM((2, G, 1, C), jnp.int32),
            pltpu.VMEM((2, G, 1, C), jnp.int32),
            pltpu.VMEM((2, C, H), jnp.float32),
            pltpu.VMEM((HB, 1, H), jnp.float32),
            pltpu.VMEM((2, 1, H), jnp.float32),
            pltpu.VMEM((HB,), jnp.int32),
            pltpu.VMEM_SHARED((N, H), jnp.float32),
            pltpu.VMEM_SHARED((HB, 1, H), jnp.float32),
            pltpu.SemaphoreType.DMA((2,)),
            pltpu.SemaphoreType.DMA((2,)),
            pltpu.SemaphoreType.DMA((3,)),
            pltpu.SemaphoreType.DMA((2,)),
            pltpu.SemaphoreType.DMA((2, 3)),
            pltpu.SemaphoreType.DMA((2, 2)),
        ],
    )
    return f(xt, x, rows2, col2)


# speedup vs baseline: 1.2636x; 1.0701x over previous
import jax
import jax.numpy as jnp
from jax import lax
from jax.experimental import pallas as pl
from jax.experimental.pallas import tpu as pltpu
from jax.experimental.pallas import tpu_sc as plsc

N = 10000
E = 160000
D = 256
H = 128
NC = 2
NS = 16
C = 80
K = (E // NS) // C
G = 25
R = 32
NCH = N // R
REM = N - NCH * R
NPASS = -(-NCH // NS)
RZ = 64
NCHZ = N // RZ
REMZ = N - NCHZ * RZ
NPASSZ = -(-NCHZ // NS)
HB = 80


def _fz_reads(c, nrows, p, cid, x_in, accv, xv, cntv, acc_sh, cnt_sh,
              rsems, make):
    nb = c * R
    o = p * R
    f = pltpu.make_async_copy if make else (
        lambda s, d, m: pltpu.async_copy(s, d, m))
    da = f(acc_sh.at[pl.ds(nb, nrows)], accv.at[pl.ds(o, nrows)],
           rsems.at[p, 0])
    dx = f(x_in.at[pl.ds(nb, nrows), pl.ds(cid * H, H)],
           xv.at[pl.ds(o, nrows)], rsems.at[p, 1])
    dc = f(cnt_sh.at[lax.shift_right_logical(nb, 7), 0], cntv.at[p, 0],
           rsems.at[p, 2])
    return da, dx, dc


def _fz_writes(c, nrows, p, cid, out, accv, xv, wsems, make):
    nb = c * R
    o = p * R
    f = pltpu.make_async_copy if make else (
        lambda s, d, m: pltpu.async_copy(s, d, m))
    wa = f(accv.at[pl.ds(o, nrows)],
           out.at[pl.ds(nb, nrows), pl.ds(cid * H, H)], wsems.at[p, 0])
    wx = f(xv.at[pl.ds(o, nrows)],
           out.at[pl.ds(nb, nrows), pl.ds(D + cid * H, H)],
           wsems.at[p, 1])
    return wa, wx


def _fz_compute(c, nrows, p, accv, xv, cntv):
    nb = c * R
    lanes0 = jnp.zeros((16,), jnp.int32)
    coff = nb & (H - 1)
    o = p * R

    def fin16(i16, _):
        cnt16 = cntv[p, 0, pl.ds(coff + i16 * 16, 16)]
        rden16 = 1.0 / jnp.maximum(cnt16, 1.0)
        mke16 = jnp.minimum(cnt16, 1.0)

        def fin(i, __):
            bi = lanes0 + i
            rden = rden16[bi]
            mke = mke16[bi]
            row = o + i16 * 16 + i
            for j in range(H // 16):
                accv[row, pl.ds(j * 16, 16)] = (
                    accv[row, pl.ds(j * 16, 16)] * rden)
                xv[row, pl.ds(j * 16, 16)] = (
                    xv[row, pl.ds(j * 16, 16)] * mke)
            return __
        lax.fori_loop(0, 16, fin, 0)
        return _
    lax.fori_loop(0, nrows // 16, fin16, 0)


def _body(xt, x_in, es2, out, ridx, cidx, rowsb,
          cntb, cntv, idv, acc_sh, cnt_sh, sems, ssems, fsems, isems,
          rsems, wsems):
    cid = lax.axis_index("c")
    sid = lax.axis_index("s")
    accv = rowsb.at[1]
    xv = rowsb.at[0]

    def zacc(i, _):
        for j in range(H // 16):
            accv[i, pl.ds(j * 16, 16)] = jnp.zeros((16,), jnp.float32)
        return _
    lax.fori_loop(0, RZ, zacc, 0)

    def zcnt(i, _):
        for j in range(H // 16):
            cntb[i, 0, pl.ds(j * 16, 16)] = jnp.zeros((16,), jnp.float32)
        return _
    lax.fori_loop(0, HB, zcnt, 0)

    for j in range(HB // 16):
        idv[pl.ds(j * 16, 16)] = lax.iota(jnp.int32, 16) + j * 16

    for q in range(NPASSZ):
        c = q * NS + sid

        @pl.when(c < NCHZ)
        def _():
            pltpu.async_copy(accv.at[pl.ds(0, RZ)],
                             acc_sh.at[pl.ds(c * RZ, RZ)], fsems.at[0])

    @pl.when(sid == NS - 1)
    def _():
        pltpu.async_copy(accv.at[pl.ds(0, REMZ)],
                         acc_sh.at[pl.ds(NCHZ * RZ, REMZ)], fsems.at[1])

    @pl.when(sid == 0)
    def _():
        pltpu.async_copy(cntb, cnt_sh, fsems.at[2])

    pltpu.async_copy(es2.at[1, sid, pl.ds(0, G)], ridx.at[0],
                     isems.at[0])
    pltpu.async_copy(es2.at[0, sid, pl.ds(0, G)], cidx.at[0], isems.at[0])

    for q in range(NPASSZ):
        c = q * NS + sid

        @pl.when(c < NCHZ)
        def _():
            pltpu.make_async_copy(accv.at[pl.ds(0, RZ)],
                                  acc_sh.at[pl.ds(c * RZ, RZ)],
                                  fsems.at[0]).wait()

    @pl.when(sid == NS - 1)
    def _():
        pltpu.make_async_copy(accv.at[pl.ds(0, REMZ)],
                              acc_sh.at[pl.ds(NCHZ * RZ, REMZ)],
                              fsems.at[1]).wait()

    @pl.when(sid == 0)
    def _():
        pltpu.make_async_copy(cntb, cnt_sh, fsems.at[2]).wait()

    pcnt, _pm = plsc.scan_count(idv[pl.ds(0, 16)])
    bias = 1 - jnp.max(pcnt)

    plsc.subcore_barrier()

    def _rfix(gb_, kk, cid_):
        for j in range(C // 16):
            v = ridx[gb_, kk, 0, pl.ds(j * 16, 16)]
            ridx[gb_, kk, 0, pl.ds(j * 16, 16)] = v + v + cid_

    for g in range(K // G):
        gb = g & 1
        pltpu.make_async_copy(es2.at[1, sid, pl.ds(g * G, G)],
                              ridx.at[gb], isems.at[gb]).wait()
        pltpu.make_async_copy(es2.at[0, sid, pl.ds(g * G, G)],
                              cidx.at[gb], isems.at[gb]).wait()
        if g + 1 < K // G:
            gb1 = (g + 1) & 1
            pltpu.async_copy(es2.at[1, sid, pl.ds((g + 1) * G, G)],
                             ridx.at[gb1], isems.at[gb1])
            pltpu.async_copy(es2.at[0, sid, pl.ds((g + 1) * G, G)],
                             cidx.at[gb1], isems.at[gb1])
        _rfix(gb, 0, cid)
        _rfix(gb, 1, cid)
        pltpu.async_copy(xt.at[ridx.at[gb, 0, 0]], rowsb.at[0], sems.at[0])

        def step(k, carry):
            b1 = (k + 1) & 1

            @pl.when(k < G - 2)
            def _():
                _rfix(gb, k + 2, cid)

            @pl.when(k >= 1)
            def _():
                pltpu.make_async_copy(
                    rowsb.at[b1], acc_sh.at[cidx.at[gb, k - 1, 0]],
                    ssems.at[b1]).wait()

            @pl.when(k < G - 1)
            def _():
                pltpu.async_copy(xt.at[ridx.at[gb, k + 1, 0]],
                                 rowsb.at[b1], sems.at[b1])

            for j in range(C // 16):
                cv = cidx[gb, k, 0, pl.ds(j * 16, 16)]
                cnt, last = plsc.scan_count(cv)
                val = (cnt + bias).astype(jnp.float32)
                plsc.addupdate_scatter(
                    cntb, [lax.shift_right_logical(cv, 7),
                           jnp.zeros((16,), jnp.int32),
                           cv & (H - 1)], val, mask=last)

            b = k & 1
            pltpu.make_async_copy(xt.at[ridx.at[gb, k, 0]], rowsb.at[b],
                                  sems.at[b]).wait()
            pltpu.async_copy(rowsb.at[b], acc_sh.at[cidx.at[gb, k, 0]],
                             ssems.at[b], add=True)
            return carry
        lax.fori_loop(0, G, step, 0)

        pltpu.make_async_copy(rowsb.at[(G - 1) & 1],
                              acc_sh.at[cidx.at[gb, G - 1, 0]],
                              ssems.at[(G - 1) & 1]).wait()

    pltpu.sync_copy(cntb, cnt_sh.at[idv], add=True)

    plsc.subcore_barrier()

    _fz_reads(sid, R, 0, cid, x_in, accv, xv, cntv, acc_sh, cnt_sh,
              rsems, False)

    def fzstep(q, carry):
        c = q * NS + sid
        p = q & 1
        p1 = 1 - p

        @pl.when(jnp.logical_and(q >= 1, c - NS < NCH))
        def _():
            for w in _fz_writes(c - NS, R, p1, cid, out, accv, xv,
                                wsems, True):
                w.wait()

        @pl.when(jnp.logical_and(q + 1 < NPASS, c + NS < NCH))
        def _():
            _fz_reads(c + NS, R, p1, cid, x_in, accv, xv, cntv, acc_sh,
                      cnt_sh, rsems, False)

        @pl.when(c < NCH)
        def _():
            for d in _fz_reads(c, R, p, cid, x_in, accv, xv, cntv,
                               acc_sh, cnt_sh, rsems, True):
                d.wait()
            _fz_compute(c, R, p, accv, xv, cntv)
            _fz_writes(c, R, p, cid, out, accv, xv, wsems, False)
        return carry
    lax.fori_loop(0, NPASS, fzstep, 0)

    qL = NPASS - 1
    cL = qL * NS + sid

    @pl.when(cL < NCH)
    def _():
        for w in _fz_writes(cL, R, qL & 1, cid, out, accv, xv, wsems,
                            True):
            w.wait()

    @pl.when(sid == NS - 1)
    def _():
        _fz_reads(NCH, REM, 0, cid, x_in, accv, xv, cntv, acc_sh,
                  cnt_sh, rsems, False)
        for d in _fz_reads(NCH, REM, 0, cid, x_in, accv, xv, cntv,
                           acc_sh, cnt_sh, rsems, True):
            d.wait()
        _fz_compute(NCH, REM, 0, accv, xv, cntv)
        _fz_writes(NCH, REM, 0, cid, out, accv, xv, wsems, False)
        for w in _fz_writes(NCH, REM, 0, cid, out, accv, xv, wsems,
                            True):
            w.wait()


@jax.jit
def kernel(x, es):
    es = es.astype(jnp.int32)
    xt = x.reshape(2 * N, H)
    es2 = es.reshape(2, NS, K, 1, C)

    mesh = plsc.VectorSubcoreMesh(core_axis_name="c", subcore_axis_name="s")
    f = pl.kernel(
        _body,
        out_type=jax.ShapeDtypeStruct((N, 2 * D), jnp.float32),
        mesh=mesh,
        compiler_params=pltpu.CompilerParams(needs_layout_passes=False),
        scratch_types=[
            pltpu.VMEM((2, G, 1, C), jnp.int32),
            pltpu.VMEM((2, G, 1, C), jnp.int32),
            pltpu.VMEM((2, C, H), jnp.float32),
            pltpu.VMEM((HB, 1, H), jnp.float32),
            pltpu.VMEM((2, 1, H), jnp.float32),
            pltpu.VMEM((HB,), jnp.int32),
            pltpu.VMEM_SHARED((N, H), jnp.float32),
            pltpu.VMEM_SHARED((HB, 1, H), jnp.float32),
            pltpu.SemaphoreType.DMA((2,)),
            pltpu.SemaphoreType.DMA((2,)),
            pltpu.SemaphoreType.DMA((3,)),
            pltpu.SemaphoreType.DMA((2,)),
            pltpu.SemaphoreType.DMA((2, 3)),
            pltpu.SemaphoreType.DMA((2, 2)),
        ],
    )
    return f(xt, x, es2)
